# Initial kernel scaffold; baseline (speedup 1.0000x reference)
import jax
import jax.numpy as jnp
from jax.experimental import pallas as pl


def kernel(logits, labels):
    raise NotImplementedError("write your pallas kernel here")


# speedup vs baseline: 1.6577x; 1.6577x over previous
import functools

import jax
import jax.numpy as jnp
import numpy as np
from jax.experimental import pallas as pl
from jax.experimental.pallas import tpu as pltpu

_SCALE = 64.0
_MARGIN = float(np.radians(28.6))
_COS_M = float(np.cos(_MARGIN))
_SIN_M = float(np.sin(_MARGIN))

_BC = 2048


def _dense_body(lbl_ref, x_ref, acc_ref, vacc_ref, *, n_classes):
    j = pl.program_id(0)
    b, bc = x_ref.shape
    cols = j * bc + jax.lax.broadcasted_iota(jnp.int32, (b, bc), 1)
    lbl = lbl_ref[...]
    x = x_ref[...]
    is_lbl = cols == lbl
    keep = jnp.logical_and(cols < n_classes, jnp.logical_not(is_lbl))
    e = jnp.where(keep, jnp.exp(x * _SCALE), 0.0)
    vpart = jnp.where(is_lbl, x, 0.0)

    @pl.when(j == 0)
    def _():
        acc_ref[...] = jnp.zeros_like(acc_ref)
        vacc_ref[...] = jnp.zeros_like(vacc_ref)

    acc_ref[...] += e.reshape(b, bc // 128, 128).sum(axis=1)
    vacc_ref[...] += vpart.reshape(b, bc // 128, 128).sum(axis=1)


def _combine_body(acc_ref, vacc_ref, out_ref):
    s = jnp.sum(acc_ref[...], axis=1, keepdims=True)
    v = jnp.sum(vacc_ref[...], axis=1, keepdims=True)
    u0 = v * _COS_M - jnp.sqrt(jnp.maximum(1.0 - v * v, 0.0)) * _SIN_M
    bits = jax.lax.bitcast_convert_type(u0, jnp.int32)
    rnd = bits + 0x0FFF + jnp.bitwise_and(jax.lax.shift_right_logical(bits, 13), 1)
    rnd = jnp.bitwise_and(rnd, jnp.int32(~0x1FFF))
    u = jax.lax.bitcast_convert_type(rnd, jnp.float32)
    t = u * _SCALE
    logz = jnp.log(s + jnp.exp(t))
    out_ref[0, 0] = jnp.mean(logz - t)


def kernel(logits, labels):
    b, n = logits.shape
    lbl2d = labels.astype(jnp.int32).reshape(b, 1)
    nb = pl.cdiv(n, _BC)
    acc, vacc = pl.pallas_call(
        functools.partial(_dense_body, n_classes=n),
        grid=(nb,),
        in_specs=[
            pl.BlockSpec((b, 1), lambda j: (0, 0)),
            pl.BlockSpec((b, _BC), lambda j: (0, j)),
        ],
        out_specs=[
            pl.BlockSpec((b, 128), lambda j: (0, 0)),
            pl.BlockSpec((b, 128), lambda j: (0, 0)),
        ],
        out_shape=[
            jax.ShapeDtypeStruct((b, 128), jnp.float32),
            jax.ShapeDtypeStruct((b, 128), jnp.float32),
        ],
    )(lbl2d, logits)
    loss = pl.pallas_call(
        _combine_body,
        out_specs=pl.BlockSpec(memory_space=pltpu.SMEM),
        out_shape=jax.ShapeDtypeStruct((1, 1), jnp.float32),
    )(acc, vacc)
    return loss.reshape(())
